# baseline (speedup 1.0000x reference)
import jax
import jax.numpy as jnp
from jax import lax
from jax.experimental import pallas as pl
from jax.experimental.pallas import tpu as pltpu
from jax.experimental.pallas import tpu_sc as plsc

D = 64
NC = 2
NS = 16
NW = NC * NS
CHUNK = 128
CH2 = CHUNK // 2


def _sc_body(seq_hbm, ids_hbm, table_hbm, out_hbm,
             ids_v0, ids_v1, idx1_0, idx1_1, idx2_0, idx2_1,
             rows1_0, rows1_1, rows2_0, rows2_1,
             seq_0, seq_1, outb_0, outb_1,
             sem_g0, sem_g1, sem_s0, sem_s1, sem_o0, sem_o1):
    ids_v = (ids_v0, ids_v1)
    idx1_v = (idx1_0, idx1_1)
    idx2_v = (idx2_0, idx2_1)
    rows1_v = (rows1_0, rows1_1)
    rows2_v = (rows2_0, rows2_1)
    seq_v = (seq_0, seq_1)
    out_v = (outb_0, outb_1)
    sem_g = (sem_g0, sem_g1)
    sem_s = (sem_s0, sem_s1)
    sem_o = (sem_o0, sem_o1)

    wid = lax.axis_index("s") * NC + lax.axis_index("c")
    n2 = seq_hbm.shape[0]
    per_w2 = n2 // NW
    steps = per_w2 // CH2
    wbase2 = wid * per_w2

    iota = lax.iota(jnp.int32, 16)
    iota4 = iota * 4
    zeros16 = iota * 0

    def prefetch(t, b):
        base2 = wbase2 + t * CH2
        pltpu.sync_copy(ids_hbm.at[pl.ds(base2 * 8, CHUNK * 4)], ids_v[b])
        for j in range(CHUNK // 16):
            off = j * 64
            idx1_v[b][pl.ds(j * 16, 16)] = plsc.load_gather(ids_v[b], [iota4 + off])
            idx2_v[b][pl.ds(j * 16, 16)] = plsc.load_gather(ids_v[b], [iota4 + (off + 2)])
        pltpu.async_copy(table_hbm.at[idx1_v[b]], rows1_v[b], sem_g[b])
        pltpu.async_copy(table_hbm.at[idx2_v[b]], rows2_v[b], sem_g[b])
        pltpu.async_copy(seq_hbm.at[pl.ds(base2, CH2)], seq_v[b], sem_s[b])

    def wait_in(b):
        pltpu.make_async_copy(table_hbm.at[idx1_v[b]], rows1_v[b], sem_g[b]).wait()
        pltpu.make_async_copy(table_hbm.at[idx2_v[b]], rows2_v[b], sem_g[b]).wait()
        pltpu.make_async_copy(seq_hbm.at[pl.ds(0, CH2)], seq_v[b], sem_s[b]).wait()

    def wait_out(b):
        pltpu.make_async_copy(out_v[b], out_hbm.at[pl.ds(0, CH2)], sem_o[b]).wait()

    def compute(t, b):
        wait_in(b)
        r1, r2, sq, ob, iv = rows1_v[b], rows2_v[b], seq_v[b], out_v[b], ids_v[b]

        @plsc.parallel_loop(0, CH2, unroll=4)
        def pair(p):
            i = p * 2
            for h in range(2):
                w1 = plsc.load_gather(iv, [zeros16 + ((i + h) * 4 + 1)]).astype(jnp.float32)
                w2 = plsc.load_gather(iv, [zeros16 + ((i + h) * 4 + 3)]).astype(jnp.float32)
                for d in range(D // 16):
                    sl = pl.ds(d * 16, 16)
                    so = pl.ds(h * 64 + d * 16, 16)
                    ob[p, so] = sq[p, so] + r1[i + h, sl] * w1 + r2[i + h, sl] * w2

        base2 = wbase2 + t * CH2
        pltpu.async_copy(ob, out_hbm.at[pl.ds(base2, CH2)], sem_o[b])

    prefetch(0, 0)
    prefetch(1, 1)
    compute(0, 0)
    prefetch(2, 0)
    compute(1, 1)
    prefetch(3, 1)

    def body(tt, _):
        t = tt * 2
        wait_out(0)
        compute(t, 0)
        prefetch(t + 2, 0)
        wait_out(1)
        compute(t + 1, 1)
        prefetch(t + 3, 1)
        return 0

    lax.fori_loop(1, steps // 2 - 1, body, 0)

    t = steps - 2
    wait_out(0)
    compute(t, 0)
    wait_out(1)
    compute(t + 1, 1)
    wait_out(0)
    wait_out(1)


def kernel(sequence, id, player_embeddings):
    b, l, d = sequence.shape
    n = b * l
    seq2 = sequence.reshape(n // 2, 2 * d)
    ids_flat = id.astype(jnp.int32).reshape(n * 4)
    table2 = jnp.pad(player_embeddings, ((0, 0), (0, 128 - d)))
    mesh = plsc.VectorSubcoreMesh(core_axis_name="c", subcore_axis_name="s")
    run = pl.kernel(
        _sc_body,
        out_type=jax.ShapeDtypeStruct((n // 2, 2 * d), jnp.float32),
        mesh=mesh,
        compiler_params=pltpu.CompilerParams(needs_layout_passes=False),
        scratch_types=[
            pltpu.VMEM((CHUNK * 4,), jnp.int32),
            pltpu.VMEM((CHUNK * 4,), jnp.int32),
            pltpu.VMEM((CHUNK,), jnp.int32),
            pltpu.VMEM((CHUNK,), jnp.int32),
            pltpu.VMEM((CHUNK,), jnp.int32),
            pltpu.VMEM((CHUNK,), jnp.int32),
            pltpu.VMEM((CHUNK, 128), jnp.float32),
            pltpu.VMEM((CHUNK, 128), jnp.float32),
            pltpu.VMEM((CHUNK, 128), jnp.float32),
            pltpu.VMEM((CHUNK, 128), jnp.float32),
            pltpu.VMEM((CH2, 128), jnp.float32),
            pltpu.VMEM((CH2, 128), jnp.float32),
            pltpu.VMEM((CH2, 128), jnp.float32),
            pltpu.VMEM((CH2, 128), jnp.float32),
            pltpu.SemaphoreType.DMA,
            pltpu.SemaphoreType.DMA,
            pltpu.SemaphoreType.DMA,
            pltpu.SemaphoreType.DMA,
            pltpu.SemaphoreType.DMA,
            pltpu.SemaphoreType.DMA,
        ],
    )
    out = run(seq2, ids_flat, table2)
    return out.reshape(b, l, d)


# speedup vs baseline: 1.1014x; 1.1014x over previous
import jax
import jax.numpy as jnp
from jax import lax
from jax.experimental import pallas as pl
from jax.experimental.pallas import tpu as pltpu
from jax.experimental.pallas import tpu_sc as plsc

D = 64
NC = 2
NS = 16
NW = NC * NS
CHUNK = 64


def _sc_body(seq_hbm, ids_hbm, table_hbm, out_hbm,
             ids_v0, ids_v1, idx1_0, idx1_1, idx2_0, idx2_1,
             rows1_0, rows1_1, rows2_0, rows2_1,
             seq_0, seq_1, outb_0, outb_1,
             sem_g0, sem_g1, sem_s0, sem_s1, sem_o0, sem_o1):
    ids_v = (ids_v0, ids_v1)
    idx1_v = (idx1_0, idx1_1)
    idx2_v = (idx2_0, idx2_1)
    rows1_v = (rows1_0, rows1_1)
    rows2_v = (rows2_0, rows2_1)
    seq_v = (seq_0, seq_1)
    out_v = (outb_0, outb_1)
    sem_g = (sem_g0, sem_g1)
    sem_s = (sem_s0, sem_s1)
    sem_o = (sem_o0, sem_o1)

    wid = lax.axis_index("s") * NC + lax.axis_index("c")
    n = seq_hbm.shape[0]
    per_w = n // NW
    steps = per_w // CHUNK
    wbase = wid * per_w

    iota = lax.iota(jnp.int32, 16)
    iota4 = iota * 4
    zeros16 = iota * 0

    def prefetch(t, b):
        base = wbase + t * CHUNK
        pltpu.sync_copy(ids_hbm.at[pl.ds(base * 4, CHUNK * 4)], ids_v[b])
        for j in range(CHUNK // 16):
            off = j * 64
            idx1_v[b][pl.ds(j * 16, 16)] = plsc.load_gather(ids_v[b], [iota4 + off])
            idx2_v[b][pl.ds(j * 16, 16)] = plsc.load_gather(ids_v[b], [iota4 + (off + 2)])
        pltpu.async_copy(table_hbm.at[idx1_v[b]], rows1_v[b], sem_g[b])
        pltpu.async_copy(table_hbm.at[idx2_v[b]], rows2_v[b], sem_g[b])
        pltpu.async_copy(seq_hbm.at[pl.ds(base, CHUNK)], seq_v[b], sem_s[b])

    def wait_in(b):
        pltpu.make_async_copy(table_hbm.at[idx1_v[b]], rows1_v[b], sem_g[b]).wait()
        pltpu.make_async_copy(table_hbm.at[idx2_v[b]], rows2_v[b], sem_g[b]).wait()
        pltpu.make_async_copy(seq_hbm.at[pl.ds(0, CHUNK)], seq_v[b], sem_s[b]).wait()

    def wait_out(b):
        pltpu.make_async_copy(out_v[b], out_hbm.at[pl.ds(0, CHUNK)], sem_o[b]).wait()

    def compute(t, b):
        wait_in(b)
        r1, r2, sq, ob, iv = rows1_v[b], rows2_v[b], seq_v[b], out_v[b], ids_v[b]

        @plsc.parallel_loop(0, CHUNK, unroll=4)
        def pos(i):
            w1 = plsc.load_gather(iv, [zeros16 + (4 * i + 1)]).astype(jnp.float32)
            w2 = plsc.load_gather(iv, [zeros16 + (4 * i + 3)]).astype(jnp.float32)
            for d in range(D // 16):
                sl = pl.ds(d * 16, 16)
                ob[i, sl] = sq[i, sl] + r1[i, sl] * w1 + r2[i, sl] * w2

        base = wbase + t * CHUNK
        pltpu.async_copy(ob, out_hbm.at[pl.ds(base, CHUNK)], sem_o[b])

    prefetch(0, 0)
    prefetch(1, 1)
    compute(0, 0)
    prefetch(2, 0)
    compute(1, 1)
    prefetch(3, 1)

    def body(tt, _):
        t = tt * 2
        wait_out(0)
        compute(t, 0)
        prefetch(t + 2, 0)
        wait_out(1)
        compute(t + 1, 1)
        prefetch(t + 3, 1)
        return 0

    lax.fori_loop(1, steps // 2 - 1, body, 0)

    t = steps - 2
    wait_out(0)
    compute(t, 0)
    wait_out(1)
    compute(t + 1, 1)
    wait_out(0)
    wait_out(1)


def kernel(sequence, id, player_embeddings):
    b, l, d = sequence.shape
    n = b * l
    seq_flat = sequence.reshape(n, d)
    ids_flat = id.astype(jnp.int32).reshape(n * 4)
    table2 = jnp.pad(player_embeddings, ((0, 0), (0, 128 - d)))
    mesh = plsc.VectorSubcoreMesh(core_axis_name="c", subcore_axis_name="s")
    run = pl.kernel(
        _sc_body,
        out_type=jax.ShapeDtypeStruct((n, d), jnp.float32),
        mesh=mesh,
        compiler_params=pltpu.CompilerParams(needs_layout_passes=False),
        scratch_types=[
            pltpu.VMEM((CHUNK * 4,), jnp.int32),
            pltpu.VMEM((CHUNK * 4,), jnp.int32),
            pltpu.VMEM((CHUNK,), jnp.int32),
            pltpu.VMEM((CHUNK,), jnp.int32),
            pltpu.VMEM((CHUNK,), jnp.int32),
            pltpu.VMEM((CHUNK,), jnp.int32),
            pltpu.VMEM((CHUNK, 128), jnp.float32),
            pltpu.VMEM((CHUNK, 128), jnp.float32),
            pltpu.VMEM((CHUNK, 128), jnp.float32),
            pltpu.VMEM((CHUNK, 128), jnp.float32),
            pltpu.VMEM((CHUNK, D), jnp.float32),
            pltpu.VMEM((CHUNK, D), jnp.float32),
            pltpu.VMEM((CHUNK, D), jnp.float32),
            pltpu.VMEM((CHUNK, D), jnp.float32),
            pltpu.SemaphoreType.DMA,
            pltpu.SemaphoreType.DMA,
            pltpu.SemaphoreType.DMA,
            pltpu.SemaphoreType.DMA,
            pltpu.SemaphoreType.DMA,
            pltpu.SemaphoreType.DMA,
        ],
    )
    out = run(seq_flat, ids_flat, table2)
    return out.reshape(b, l, d)


# speedup vs baseline: 3.8371x; 3.4839x over previous
import jax
import jax.numpy as jnp
from jax import lax
from jax.experimental import pallas as pl
from jax.experimental.pallas import tpu as pltpu
from jax.experimental.pallas import tpu_sc as plsc

D = 64
NC = 2
NS = 16
NW = NC * NS
BW = 128
LG = 8
BC = 8
CPOS = LG * BC
NCH = BW // BC


def _sc_body(seq_hbm, ids_hbm, table_hbm, out_hbm,
             slab_v,
             idx1_0, idx1_1, idx2_0, idx2_1, w1_0, w1_1, w2_0, w2_1,
             rows1_0, rows1_1, rows2_0, rows2_1,
             seq_0, seq_1, outb_0, outb_1,
             sem_g0, sem_g1, sem_s0, sem_s1, sem_o0, sem_o1):
    idx1_v = (idx1_0, idx1_1)
    idx2_v = (idx2_0, idx2_1)
    w1_v = (w1_0, w1_1)
    w2_v = (w2_0, w2_1)
    rows1_v = (rows1_0, rows1_1)
    rows2_v = (rows2_0, rows2_1)
    seq_v = (seq_0, seq_1)
    out_v = (outb_0, outb_1)
    sem_g = (sem_g0, sem_g1)
    sem_s = (sem_s0, sem_s1)
    sem_o = (sem_o0, sem_o1)

    wid = lax.axis_index("s") * NC + lax.axis_index("c")
    wb = wid * BW
    ngroups = ids_hbm.shape[0] // LG

    iota = lax.iota(jnp.int32, 16)
    zeros16 = iota * 0
    ones16 = zeros16 + 1
    li_lo = lax.shift_right_logical(iota, 3)
    bi16 = lax.bitwise_and(iota, zeros16 + 7)

    def stage_slab(g):
        pltpu.sync_copy(ids_hbm.at[pl.ds(g * LG, LG), :, :, pl.ds(wb, BW)],
                        slab_v)

    def prefetch(g, k, b):
        for v in range(CPOS // 16):
            li = li_lo + 2 * v
            bv = bi16 + k * BC
            sl = pl.ds(v * 16, 16)
            idx1_v[b][sl] = plsc.load_gather(slab_v, [li, zeros16, zeros16, bv])
            w1_v[b][sl] = plsc.load_gather(slab_v, [li, zeros16, ones16, bv])
            idx2_v[b][sl] = plsc.load_gather(slab_v, [li, ones16, zeros16, bv])
            w2_v[b][sl] = plsc.load_gather(slab_v, [li, ones16, ones16, bv])
        pltpu.async_copy(table_hbm.at[idx1_v[b]], rows1_v[b], sem_g[b])
        pltpu.async_copy(table_hbm.at[idx2_v[b]], rows2_v[b], sem_g[b])
        b0 = wb + k * BC
        l0 = g * LG
        pltpu.async_copy(seq_hbm.at[pl.ds(b0, BC), pl.ds(l0, LG)], seq_v[b],
                         sem_s[b])

    def wait_in(b):
        pltpu.make_async_copy(table_hbm.at[idx1_v[b]], rows1_v[b], sem_g[b]).wait()
        pltpu.make_async_copy(table_hbm.at[idx2_v[b]], rows2_v[b], sem_g[b]).wait()
        pltpu.make_async_copy(seq_hbm.at[pl.ds(0, BC), pl.ds(0, LG)], seq_v[b],
                              sem_s[b]).wait()

    def wait_out(b):
        pltpu.make_async_copy(out_v[b], out_hbm.at[pl.ds(0, BC), pl.ds(0, LG)],
                              sem_o[b]).wait()

    def compute(g, k, b):
        wait_in(b)

        @pl.when(g * NCH + k >= 2)
        def _():
            wait_out(b)

        r1, r2, sq, ob = rows1_v[b], rows2_v[b], seq_v[b], out_v[b]
        wv1, wv2 = w1_v[b], w2_v[b]

        @plsc.parallel_loop(0, CPOS, unroll=4)
        def pos(p):
            bi = lax.bitwise_and(p, BC - 1)
            li = lax.shift_right_logical(p, 3)
            ws1 = plsc.load_gather(wv1, [zeros16 + p]).astype(jnp.float32)
            ws2 = plsc.load_gather(wv2, [zeros16 + p]).astype(jnp.float32)
            for d in range(D // 16):
                sl = pl.ds(d * 16, 16)
                ob[bi, li, sl] = (sq[bi, li, sl] + r1[p, sl] * ws1
                                  + r2[p, sl] * ws2)

        b0 = wb + k * BC
        l0 = g * LG
        pltpu.async_copy(ob, out_hbm.at[pl.ds(b0, BC), pl.ds(l0, LG)], sem_o[b])

    def body(g, _):
        stage_slab(g)
        prefetch(g, 0, 0)
        prefetch(g, 1, 1)

        def inner(k2, _):
            k = k2 * 2
            compute(g, k, 0)
            prefetch(g, k + 2, 0)
            compute(g, k + 1, 1)
            prefetch(g, k + 3, 1)
            return 0

        lax.fori_loop(0, NCH // 2 - 1, inner, 0)
        compute(g, NCH - 2, 0)
        compute(g, NCH - 1, 1)
        return 0

    lax.fori_loop(0, ngroups, body, 0)

    wait_out(0)
    wait_out(1)


def kernel(sequence, id, player_embeddings):
    b, l, d = sequence.shape
    ids_t = jnp.transpose(id.astype(jnp.int32), (1, 2, 3, 0))
    table2 = jnp.pad(player_embeddings, ((0, 0), (0, 128 - d)))
    mesh = plsc.VectorSubcoreMesh(core_axis_name="c", subcore_axis_name="s")
    run = pl.kernel(
        _sc_body,
        out_type=jax.ShapeDtypeStruct((b, l, d), jnp.float32),
        mesh=mesh,
        compiler_params=pltpu.CompilerParams(needs_layout_passes=False),
        scratch_types=[
            pltpu.VMEM((LG, 2, 2, BW), jnp.int32),
            pltpu.VMEM((CPOS,), jnp.int32),
            pltpu.VMEM((CPOS,), jnp.int32),
            pltpu.VMEM((CPOS,), jnp.int32),
            pltpu.VMEM((CPOS,), jnp.int32),
            pltpu.VMEM((CPOS,), jnp.int32),
            pltpu.VMEM((CPOS,), jnp.int32),
            pltpu.VMEM((CPOS,), jnp.int32),
            pltpu.VMEM((CPOS,), jnp.int32),
            pltpu.VMEM((CPOS, 128), jnp.float32),
            pltpu.VMEM((CPOS, 128), jnp.float32),
            pltpu.VMEM((CPOS, 128), jnp.float32),
            pltpu.VMEM((CPOS, 128), jnp.float32),
            pltpu.VMEM((BC, LG, D), jnp.float32),
            pltpu.VMEM((BC, LG, D), jnp.float32),
            pltpu.VMEM((BC, LG, D), jnp.float32),
            pltpu.VMEM((BC, LG, D), jnp.float32),
            pltpu.SemaphoreType.DMA,
            pltpu.SemaphoreType.DMA,
            pltpu.SemaphoreType.DMA,
            pltpu.SemaphoreType.DMA,
            pltpu.SemaphoreType.DMA,
            pltpu.SemaphoreType.DMA,
        ],
    )
    return run(sequence, ids_t, table2)
